# baseline (speedup 1.0000x reference)
import functools

import jax
import jax.numpy as jnp
from jax import lax
from jax.experimental import pallas as pl
from jax.experimental.pallas import tpu as pltpu
from jax.experimental.pallas import tpu_sc as plsc

VOCAB = 28996
EMBED_DIM = 64
NSEQ = 4096
SEQLEN = 200

NC, NS, L = 2, 16, 16
NW = NC * NS

BPW = NSEQ // NW
PAIR_ROWS = VOCAB * EMBED_DIM // 128

NBUF = 4
N_GROUPS = SEQLEN // NBUF


@functools.cache
def _build_gather_kernel():
    mesh = plsc.VectorSubcoreMesh(core_axis_name="c", subcore_axis_name="s")
    return functools.partial(
        pl.kernel,
        out_type=jax.ShapeDtypeStruct((SEQLEN, EMBED_DIM, NSEQ), jnp.float32),
        mesh=mesh,
        compiler_params=pltpu.CompilerParams(needs_layout_passes=False),
        scratch_types=[
            pltpu.VMEM((SEQLEN, BPW), jnp.int32),
            pltpu.VMEM((NBUF, BPW), jnp.int32),
            pltpu.VMEM((NBUF, BPW, 128), jnp.float32),
            pltpu.VMEM((NBUF, EMBED_DIM, BPW), jnp.float32),
            [pltpu.SemaphoreType.DMA] * NBUF,
            [pltpu.SemaphoreType.DMA] * NBUF,
        ],
    )(_gather_body)


def _gather_body(xt_hbm, tab_hbm, out_hbm, idx_v, pidx_r, rows_r, tb_r, gsems, osems):
    wid = lax.axis_index("s") * NC + lax.axis_index("c")

    pltpu.sync_copy(xt_hbm.at[wid], idx_v)

    iota = lax.iota(jnp.int32, 16)
    row_ids = [iota + g * 16 for g in range(8)]

    def prep(t, b):
        for c in range(8):
            v = idx_v[t, pl.ds(c * 16, 16)]
            pidx_r[b, pl.ds(c * 16, 16)] = v >> 1

    def gather(b):
        return pltpu.make_async_copy(
            tab_hbm.at[pidx_r.at[b]], rows_r.at[b], gsems[b]
        )

    def out_dma(t, b):
        return pltpu.make_async_copy(
            tb_r.at[b],
            out_hbm.at[t, :, pl.ds(wid * BPW, BPW)],
            osems[b],
        )

    def transpose(t, b):
        cbs = [(idx_v[t, pl.ds(g * 16, 16)] & 1) << 6 for g in range(8)]

        def ebody(e, carry):
            for g in range(8):
                col = cbs[g] + e
                v = plsc.load_gather(rows_r.at[b], [row_ids[g], col])
                tb_r[b, e, pl.ds(g * 16, 16)] = v
            return carry

        lax.fori_loop(0, EMBED_DIM, ebody, 0, unroll=4)

    for b in range(NBUF):
        prep(b, b)
        gather(b).start()
    for b in range(NBUF):
        gather(b).wait()
        transpose(b, b)
        out_dma(b, b).start()

    def group(g, carry):
        for b in range(NBUF):
            t = g * NBUF + b
            out_dma(t - NBUF, b).wait()
            prep(t, b)
            gather(b).start()
        for b in range(NBUF):
            t = g * NBUF + b
            gather(b).wait()
            transpose(t, b)
            out_dma(t, b).start()
        return carry

    lax.fori_loop(1, N_GROUPS, group, 0)

    for b in range(NBUF):
        out_dma((N_GROUPS - 1) * NBUF + b, b).wait()


def kernel(x, table):
    xt = x.T.reshape(SEQLEN, NW, BPW).transpose(1, 0, 2)
    tab2 = table.reshape(PAIR_ROWS, 128)
    out_t = _build_gather_kernel()(xt, tab2)
    return jnp.transpose(out_t, (2, 0, 1))
